# Initial kernel scaffold; baseline (speedup 1.0000x reference)
import functools

import jax
import jax.numpy as jnp
from jax.experimental import pallas as pl

B, S, H = 2, 4096, 1024
E, TOPK = 8, 2
INTER = 4096
MOE_INTER = INTER // TOPK
T = B * S
NE = E + 2

TT_R = 2048
TT = 4096
MM = 512


def _router_kernel(x_ref, w9_ref, logits_ref, wall_ref):
    x = x_ref[...]
    w9 = w9_ref[...]
    l9 = jax.lax.dot_general(x, w9, (((1,), (1,)), ((), ())),
                             preferred_element_type=jnp.float32)
    logits = l9[:, :E]
    logits_ref[...] = logits
    sig = jax.nn.sigmoid(l9[:, E:E + 1])
    m = jnp.max(logits, axis=1, keepdims=True)
    p = jnp.exp(logits - m)
    p = p / jnp.sum(p, axis=1, keepdims=True)
    w1 = jnp.max(p, axis=1, keepdims=True)
    i1 = jnp.argmax(p, axis=1)[:, None]
    cols8 = jax.lax.broadcasted_iota(jnp.int32, (TT_R, E), 1)
    p2 = jnp.where(cols8 == i1, -jnp.inf, p)
    w2 = jnp.max(p2, axis=1, keepdims=True)
    i2 = jnp.argmax(p2, axis=1)[:, None]
    cols10 = jax.lax.broadcasted_iota(jnp.int32, (TT_R, NE), 1)
    wall = jnp.where(cols10 == i1, w1, 0.0) + jnp.where(cols10 == i2, w2, 0.0)
    wall = jnp.where(cols10 >= E, sig, wall)
    wall_ref[...] = wall


def _experts_kernel(x_ref, wall_ref, wg_ref, wu_ref, wd_ref, out_ref):
    e = pl.program_id(1)
    m = pl.program_id(2)

    @pl.when((e == 0) & (m == 0))
    def _():
        out_ref[...] = jnp.zeros_like(out_ref)

    x = x_ref[...]
    wg = wg_ref[0]
    wu = wu_ref[0]
    wd = wd_ref[0]
    g = jax.lax.dot_general(x, wg, (((1,), (1,)), ((), ())),
                            preferred_element_type=jnp.float32)
    u = jax.lax.dot_general(x, wu, (((1,), (1,)), ((), ())),
                            preferred_element_type=jnp.float32)
    w = wall_ref[0]
    h = (g * jax.nn.sigmoid(g) * u * w).astype(jnp.bfloat16)
    y = jax.lax.dot_general(h, wd, (((1,), (1,)), ((), ())),
                            preferred_element_type=jnp.float32)
    out_ref[...] += y


def kernel(hidden_states, gate_W, expert_gate_W, expert_up_W, expert_down_W,
           shared_gate_W, shared_up_W, shared_down_W, shared_expert_gate_W):
    x32 = hidden_states.reshape(T, H)
    xbf = x32.astype(jnp.bfloat16)

    w9 = jnp.concatenate([gate_W, shared_expert_gate_W], axis=0)

    logits, wall = pl.pallas_call(
        _router_kernel,
        grid=(T // TT_R,),
        in_specs=[
            pl.BlockSpec((TT_R, H), lambda t: (t, 0)),
            pl.BlockSpec((E + 1, H), lambda t: (0, 0)),
        ],
        out_specs=[
            pl.BlockSpec((TT_R, E), lambda t: (t, 0)),
            pl.BlockSpec((TT_R, NE), lambda t: (t, 0)),
        ],
        out_shape=[
            jax.ShapeDtypeStruct((T, E), jnp.float32),
            jax.ShapeDtypeStruct((T, NE), jnp.float32),
        ],
    )(x32, w9)

    wall3 = wall.T[:, :, None]

    bf = jnp.bfloat16
    wg_all = jnp.concatenate(
        [expert_gate_W.astype(bf), shared_gate_W.astype(bf).reshape(2, MOE_INTER, H)], axis=0)
    wu_all = jnp.concatenate(
        [expert_up_W.astype(bf), shared_up_W.astype(bf).reshape(2, MOE_INTER, H)], axis=0)
    sd = shared_down_W.astype(bf).reshape(H, 2, MOE_INTER).transpose(1, 0, 2)
    wd_all = jnp.concatenate([expert_down_W.astype(bf), sd], axis=0)

    out = pl.pallas_call(
        _experts_kernel,
        grid=(T // TT, NE, MOE_INTER // MM),
        in_specs=[
            pl.BlockSpec((TT, H), lambda t, e, m: (t, 0)),
            pl.BlockSpec((1, TT, 1), lambda t, e, m: (e, t, 0)),
            pl.BlockSpec((1, MM, H), lambda t, e, m: (e, m, 0)),
            pl.BlockSpec((1, MM, H), lambda t, e, m: (e, m, 0)),
            pl.BlockSpec((1, H, MM), lambda t, e, m: (e, 0, m)),
        ],
        out_specs=pl.BlockSpec((TT, H), lambda t, e, m: (t, 0)),
        out_shape=jax.ShapeDtypeStruct((T, H), jnp.float32),
    )(xbf, wall3, wg_all, wu_all, wd_all)

    return out.reshape(B, S, H), logits


# speedup vs baseline: 1.0627x; 1.0627x over previous
import functools

import jax
import jax.numpy as jnp
from jax.experimental import pallas as pl

B, S, H = 2, 4096, 1024
E, TOPK = 8, 2
INTER = 4096
MOE_INTER = INTER // TOPK
T = B * S
NE = E + 2

TT_R = 2048
TT = 2048
MM = 512


def _router_kernel(x_ref, w9_ref, logits_ref, wall_ref):
    x = x_ref[...]
    w9 = w9_ref[...]
    l9 = jax.lax.dot_general(x, w9, (((1,), (1,)), ((), ())),
                             preferred_element_type=jnp.float32)
    logits = l9[:, :E]
    logits_ref[...] = logits
    sig = jax.nn.sigmoid(l9[:, E:E + 1])
    m = jnp.max(logits, axis=1, keepdims=True)
    p = jnp.exp(logits - m)
    p = p / jnp.sum(p, axis=1, keepdims=True)
    w1 = jnp.max(p, axis=1, keepdims=True)
    i1 = jnp.argmax(p, axis=1)[:, None]
    cols8 = jax.lax.broadcasted_iota(jnp.int32, (TT_R, E), 1)
    p2 = jnp.where(cols8 == i1, -jnp.inf, p)
    w2 = jnp.max(p2, axis=1, keepdims=True)
    i2 = jnp.argmax(p2, axis=1)[:, None]
    cols10 = jax.lax.broadcasted_iota(jnp.int32, (TT_R, NE), 1)
    wall = jnp.where(cols10 == i1, w1, 0.0) + jnp.where(cols10 == i2, w2, 0.0)
    wall = jnp.where(cols10 >= E, sig, wall)
    wall_ref[...] = wall


def _experts_kernel(x_ref, wall_ref, wg_ref, wu_ref, wd_ref, out_ref):
    e = pl.program_id(1)
    m = pl.program_id(2)

    @pl.when((e == 0) & (m == 0))
    def _():
        out_ref[...] = jnp.zeros_like(out_ref)

    x = x_ref[...]
    wg = wg_ref[0]
    wu = wu_ref[0]
    wd = wd_ref[0]
    g = jax.lax.dot_general(x, wg, (((1,), (1,)), ((), ())),
                            preferred_element_type=jnp.float32)
    u = jax.lax.dot_general(x, wu, (((1,), (1,)), ((), ())),
                            preferred_element_type=jnp.float32)
    w = wall_ref[0]
    h = (g * jax.nn.sigmoid(g) * u * w).astype(jnp.bfloat16)
    y = jax.lax.dot_general(h, wd, (((1,), (1,)), ((), ())),
                            preferred_element_type=jnp.float32)
    out_ref[...] += y


def kernel(hidden_states, gate_W, expert_gate_W, expert_up_W, expert_down_W,
           shared_gate_W, shared_up_W, shared_down_W, shared_expert_gate_W):
    x32 = hidden_states.reshape(T, H)
    xbf = x32.astype(jnp.bfloat16)

    w9 = jnp.concatenate([gate_W, shared_expert_gate_W], axis=0)

    logits, wall = pl.pallas_call(
        _router_kernel,
        grid=(T // TT_R,),
        in_specs=[
            pl.BlockSpec((TT_R, H), lambda t: (t, 0)),
            pl.BlockSpec((E + 1, H), lambda t: (0, 0)),
        ],
        out_specs=[
            pl.BlockSpec((TT_R, E), lambda t: (t, 0)),
            pl.BlockSpec((TT_R, NE), lambda t: (t, 0)),
        ],
        out_shape=[
            jax.ShapeDtypeStruct((T, E), jnp.float32),
            jax.ShapeDtypeStruct((T, NE), jnp.float32),
        ],
    )(x32, w9)

    wall3 = wall.T[:, :, None]

    bf = jnp.bfloat16
    wg_all = jnp.concatenate(
        [expert_gate_W.astype(bf), shared_gate_W.astype(bf).reshape(2, MOE_INTER, H)], axis=0)
    wu_all = jnp.concatenate(
        [expert_up_W.astype(bf), shared_up_W.astype(bf).reshape(2, MOE_INTER, H)], axis=0)
    sd = shared_down_W.astype(bf).reshape(H, 2, MOE_INTER).transpose(1, 0, 2)
    wd_all = jnp.concatenate([expert_down_W.astype(bf), sd], axis=0)

    out = pl.pallas_call(
        _experts_kernel,
        grid=(T // TT, NE, MOE_INTER // MM),
        in_specs=[
            pl.BlockSpec((TT, H), lambda t, e, m: (t, 0)),
            pl.BlockSpec((1, TT, 1), lambda t, e, m: (e, t, 0)),
            pl.BlockSpec((1, MM, H), lambda t, e, m: (e, m, 0)),
            pl.BlockSpec((1, MM, H), lambda t, e, m: (e, m, 0)),
            pl.BlockSpec((1, H, MM), lambda t, e, m: (e, 0, m)),
        ],
        out_specs=pl.BlockSpec((TT, H), lambda t, e, m: (t, 0)),
        out_shape=jax.ShapeDtypeStruct((T, H), jnp.float32),
    )(xbf, wall3, wg_all, wu_all, wd_all)

    return out.reshape(B, S, H), logits
